# baseline (speedup 1.0000x reference)
import functools

import jax
import jax.numpy as jnp
from jax import lax
from jax.experimental import pallas as pl
from jax.experimental.pallas import tpu as pltpu
from jax.experimental.pallas import tpu_sc as plsc

VOCAB = 1000000
EMBED = 64
BATCH = 4096
HIST = 200
TOTAL = BATCH * HIST

NC, NS = 2, 16
NW = NC * NS
BPW = TOTAL // NW
BATW = BATCH // NW
IDXW = 80
NBAT = 2
CHUNK = NBAT * HIST
GATHERS = CHUNK // IDXW
NCHUNKS = BPW // CHUNK
SCALE = 8.0

_mesh = plsc.VectorSubcoreMesh(core_axis_name="c", subcore_axis_name="s")


@functools.partial(
    pl.kernel,
    out_type=jax.ShapeDtypeStruct((BATCH, HIST, EMBED), jnp.float32),
    mesh=_mesh,
    scratch_types=[
        pltpu.VMEM((BPW,), jnp.int32),
        pltpu.VMEM((2, CHUNK, EMBED), jnp.float32),
        pltpu.SemaphoreType.DMA,
        pltpu.SemaphoreType.DMA,
    ],
    compiler_params=pltpu.CompilerParams(use_tc_tiling_on_sc=False),
)
def _gather_scale(x_hbm, emb_hbm, out_hbm, idx_v, rows_v, sem0, sem1):
    wid = lax.axis_index("s") * NC + lax.axis_index("c")
    base = wid * BPW
    bat0 = wid * BATW
    sems = (sem0, sem1)

    pltpu.sync_copy(x_hbm.at[pl.ds(pl.multiple_of(base, BPW), BPW)], idx_v)

    def fire(buf, chunk):
        for g in range(GATHERS):
            off = pl.multiple_of(chunk * CHUNK + g * IDXW, IDXW)
            pltpu.async_copy(
                emb_hbm.at[idx_v.at[pl.ds(off, IDXW)]],
                rows_v.at[buf, pl.ds(g * IDXW, IDXW)],
                sems[buf],
            )

    def drain(buf):
        for g in range(GATHERS):
            pltpu.make_async_copy(
                emb_hbm.at[idx_v.at[pl.ds(g * IDXW, IDXW)]],
                rows_v.at[buf, pl.ds(g * IDXW, IDXW)],
                sems[buf],
            ).wait()

    fire(0, 0)

    @pl.loop(0, NCHUNKS, step=2)
    def _steps(ci):
        for b in (0, 1):
            cur = ci + b

            @pl.when(cur + 1 < NCHUNKS)
            def _():
                fire(1 - b, cur + 1)

            drain(b)

            @plsc.parallel_loop(0, CHUNK, step=1, unroll=8)
            def _scale(r):
                for j in range(EMBED // 16):
                    sl = pl.ds(j * 16, 16)
                    rows_v[b, r, sl] = rows_v[b, r, sl] * SCALE

            for k in range(NBAT):
                pltpu.sync_copy(
                    rows_v.at[b, pl.ds(k * HIST, HIST)],
                    out_hbm.at[bat0 + cur * NBAT + k],
                )


def kernel(x, embedding):
    x2 = x.reshape(TOTAL)
    return _gather_scale(x2, embedding)
